# Initial kernel scaffold; baseline (speedup 1.0000x reference)
import jax
import jax.numpy as jnp
from jax.experimental import pallas as pl


def kernel(x, edge_index, W1, b1, W2, b2, W3, b3):
    raise NotImplementedError("write your pallas kernel here")


# speedup vs baseline: 3.5562x; 3.5562x over previous
import functools
import jax
import jax.numpy as jnp
from jax import lax
from jax.experimental import pallas as pl
from jax.experimental.pallas import tpu as pltpu
from jax.experimental.pallas import tpu_sc as plsc

D = 128
CHUNK = 128
NC = 2
NS = 16
NW = NC * NS


def _sc_mesh():
    return plsc.VectorSubcoreMesh(core_axis_name="c", subcore_axis_name="s")


def _make_deg_kernel(n_pad, e_per_w):
    rows_per_tec = n_pad // NS
    n_chunks = e_per_w // CHUNK

    @functools.partial(
        pl.kernel,
        mesh=_sc_mesh(),
        out_type=jax.ShapeDtypeStruct((NC, n_pad, D), jnp.float32),
        scratch_types=[
            pltpu.VMEM((CHUNK,), jnp.int32),
            pltpu.VMEM((CHUNK,), jnp.int32),
            pltpu.VMEM((CHUNK, D), jnp.float32),
            pltpu.VMEM((CHUNK, D), jnp.float32),
            pltpu.VMEM_SHARED((n_pad, D), jnp.float32),
        ],
    )
    def deg_kernel(src_hbm, dst_hbm, usrc_hbm, udst_hbm, zeros_hbm,
                   deg_out,
                   src_v, dst_v, usrc_v, udst_v, deg_sh):
        c = lax.axis_index("c")
        s = lax.axis_index("s")
        w = c * NS + s
        my_rows = pl.ds(s * rows_per_tec, rows_per_tec)
        pltpu.sync_copy(zeros_hbm, deg_sh.at[my_rows])
        pltpu.sync_copy(usrc_hbm, usrc_v)
        pltpu.sync_copy(udst_hbm, udst_v)
        plsc.subcore_barrier()

        def body(i, carry):
            base = pl.multiple_of(w * e_per_w + i * CHUNK, CHUNK)
            pltpu.sync_copy(src_hbm.at[pl.ds(base, CHUNK)], src_v)
            pltpu.sync_copy(dst_hbm.at[pl.ds(base, CHUNK)], dst_v)
            pltpu.sync_copy(usrc_v, deg_sh.at[src_v], add=True)
            pltpu.sync_copy(udst_v, deg_sh.at[dst_v], add=True)
            return carry

        lax.fori_loop(0, n_chunks, body, 0)
        plsc.subcore_barrier()
        pltpu.sync_copy(deg_sh.at[my_rows], deg_out.at[c, my_rows])

    return deg_kernel


def _make_edge_kernel(n_pad, e_per_w):
    rows_per_tec = n_pad // NS
    n_chunks = e_per_w // CHUNK

    @functools.partial(
        pl.kernel,
        mesh=_sc_mesh(),
        out_type=jax.ShapeDtypeStruct((NC, n_pad, D), jnp.float32),
        scratch_types=[
            pltpu.VMEM((CHUNK,), jnp.int32),
            pltpu.VMEM((CHUNK,), jnp.int32),
            pltpu.VMEM((CHUNK, D), jnp.float32),
            pltpu.VMEM_SHARED((n_pad, D), jnp.float32),
            pltpu.SemaphoreType.DMA,
        ],
    )
    def edge_kernel(h_hbm, src_hbm, dst_hbm, zeros_hbm, agg_out,
                    src_v, dst_v, rows_v, agg_sh, sem):
        c = lax.axis_index("c")
        s = lax.axis_index("s")
        w = c * NS + s
        my_rows = pl.ds(s * rows_per_tec, rows_per_tec)
        pltpu.sync_copy(zeros_hbm, agg_sh.at[my_rows])
        plsc.subcore_barrier()

        def body(i, carry):
            base = pl.multiple_of(w * e_per_w + i * CHUNK, CHUNK)
            pltpu.sync_copy(src_hbm.at[pl.ds(base, CHUNK)], src_v)
            pltpu.sync_copy(dst_hbm.at[pl.ds(base, CHUNK)], dst_v)
            pltpu.async_copy(h_hbm.at[src_v], rows_v, sem).wait()
            pltpu.sync_copy(rows_v, agg_sh.at[dst_v], add=True)
            return carry

        lax.fori_loop(0, n_chunks, body, 0)
        plsc.subcore_barrier()
        pltpu.sync_copy(agg_sh.at[my_rows], agg_out.at[c, my_rows])

    return edge_kernel


def _norm_col(deg_ref):
    deg = deg_ref[0, :, 0:1] + deg_ref[1, :, 0:1]
    return lax.rsqrt(jnp.maximum(deg, 1.0))


def _mm_scale_body(x_ref, w_ref, dout_ref, o_ref):
    ns = _norm_col(dout_ref)
    o_ref[...] = jnp.dot(x_ref[...], w_ref[...],
                         preferred_element_type=jnp.float32) * ns


def _boundary_body(agg_ref, din_ref, dout_ref, b_ref, w_ref, o_ref):
    agg = agg_ref[0] + agg_ref[1]
    nd = _norm_col(din_ref)
    t = jnp.maximum(agg * nd + b_ref[...], 0.0)
    ns = _norm_col(dout_ref)
    o_ref[...] = jnp.dot(t, w_ref[...],
                         preferred_element_type=jnp.float32) * ns


def _final_body(agg_ref, din_ref, b_ref, o_ref):
    agg = agg_ref[0] + agg_ref[1]
    nd = _norm_col(din_ref)
    o_ref[...] = agg * nd + b_ref[...]


def _tc_grid_call(body, n_pad, r, ins, in_specs):
    grid = n_pad // r
    return pl.pallas_call(
        body,
        grid=(grid,),
        in_specs=in_specs,
        out_specs=pl.BlockSpec((r, D), lambda i: (i, 0)),
        out_shape=jax.ShapeDtypeStruct((n_pad, D), jnp.float32),
    )(*ins)


def _spec_rows(r):
    return pl.BlockSpec((r, D), lambda i: (i, 0))


def _spec_deg(r):
    return pl.BlockSpec((NC, r, 16), lambda i: (0, i, 0))


def _spec_agg(r):
    return pl.BlockSpec((NC, r, D), lambda i: (0, i, 0))


def _spec_full(shape):
    nd = len(shape)
    return pl.BlockSpec(shape, lambda i: (0,) * nd)


def kernel(x, edge_index, W1, b1, W2, b2, W3, b3):
    n = x.shape[0]
    e = edge_index.shape[1]

    r = 1024
    n_pad = ((n + 1 + r - 1) // r) * r
    e_per_w = ((e + NW * CHUNK - 1) // (NW * CHUNK)) * CHUNK
    e_pad = e_per_w * NW

    pad_idx = jnp.full((e_pad - e,), n, dtype=jnp.int32)
    src = jnp.concatenate([edge_index[0], pad_idx])
    dst = jnp.concatenate([edge_index[1], pad_idx])
    xp = jnp.pad(x, ((0, n_pad - n), (0, 0)))

    rows_per_tec = n_pad // NS
    zeros_rows = jnp.zeros((rows_per_tec, D), jnp.float32)
    half = D // 2
    col = jnp.arange(D)
    u_src = jnp.broadcast_to((col < half).astype(jnp.float32), (CHUNK, D))
    u_dst = jnp.broadcast_to((col >= half).astype(jnp.float32), (CHUNK, D))

    deg_kernel = _make_deg_kernel(n_pad, e_per_w)
    edge_kernel = _make_edge_kernel(n_pad, e_per_w)
    deg_tbl = deg_kernel(src, dst, u_src, u_dst, zeros_rows)
    dsrc = lax.slice(deg_tbl, (0, 0, 0), (NC, n_pad, 16))
    ddst = lax.slice(deg_tbl, (0, 0, half), (NC, n_pad, half + 16))

    b1r = b1.reshape(1, D)
    b2r = b2.reshape(1, D)
    b3r = b3.reshape(1, D)

    h1 = _tc_grid_call(
        _mm_scale_body, n_pad, r,
        [xp, W1, dsrc],
        [_spec_rows(r), _spec_full((D, D)), _spec_deg(r)],
    )
    a1 = edge_kernel(h1, src, dst, zeros_rows)

    h2 = _tc_grid_call(
        _boundary_body, n_pad, r,
        [a1, ddst, dsrc, b1r, W2],
        [_spec_agg(r), _spec_deg(r), _spec_deg(r), _spec_full((1, D)),
         _spec_full((D, D))],
    )
    a2 = edge_kernel(h2, src, dst, zeros_rows)

    h3 = _tc_grid_call(
        _boundary_body, n_pad, r,
        [a2, ddst, dsrc, b2r, W3],
        [_spec_agg(r), _spec_deg(r), _spec_deg(r), _spec_full((1, D)),
         _spec_full((D, D))],
    )
    a3 = edge_kernel(h3, src, dst, zeros_rows)

    out = _tc_grid_call(
        _final_body, n_pad, r,
        [a3, ddst, b3r],
        [_spec_agg(r), _spec_deg(r), _spec_full((1, D))],
    )
    return out[:n]
